# initial kernel scaffold (unmeasured)
import jax
import jax.numpy as jnp
from jax import lax
from jax.experimental import pallas as pl
from jax.experimental.pallas import tpu as pltpu

M = 8192
D = 4096
HALF = M // 2
QTR = HALF // 2
CH = 256


def kernel(partial, gamma):
    p2d = partial.reshape(M, D)
    g2d = gamma.reshape(1, D)

    def body(p_ref, g_ref, out_ref, rbuf, send_buf, stage, out_stage,
             send_sem_y, recv_sem_y, send_sem_x, recv_sem_x, copy_sem):
        my_x = lax.axis_index("x")
        my_y = lax.axis_index("y")
        y_nbr = (my_x, 1 - my_y)
        x_nbr = (1 - my_x, my_y)

        barrier = pltpu.get_barrier_semaphore()
        for nbr in (y_nbr, x_nbr):
            pl.semaphore_signal(barrier, inc=1, device_id=nbr,
                                device_id_type=pl.DeviceIdType.MESH)
        pl.semaphore_wait(barrier, 2)

        send_base = (1 - my_y) * HALF + my_x * QTR
        for c in range(QTR // CH):
            cp = pltpu.make_async_copy(
                p_ref.at[pl.ds(send_base + c * CH, CH), :], stage, copy_sem)
            cp.start()
            cp.wait()
            send_buf[pl.ds(c * CH, CH), :] = stage[...].astype(jnp.bfloat16)

        rdma_y = pltpu.make_async_remote_copy(
            src_ref=send_buf,
            dst_ref=rbuf.at[pl.ds(my_x * QTR, QTR), :],
            send_sem=send_sem_y, recv_sem=recv_sem_y,
            device_id=y_nbr, device_id_type=pl.DeviceIdType.MESH)
        rdma_y.start()
        rdma_y.wait()

        rdma_x = pltpu.make_async_remote_copy(
            src_ref=rbuf.at[pl.ds(my_x * QTR, QTR), :],
            dst_ref=rbuf.at[pl.ds(my_x * QTR, QTR), :],
            send_sem=send_sem_x, recv_sem=recv_sem_x,
            device_id=x_nbr, device_id_type=pl.DeviceIdType.MESH)
        rdma_x.start()
        rdma_x.wait()

        out_base = my_y * HALF
        for c in range(HALF // CH):
            cp = pltpu.make_async_copy(
                p_ref.at[pl.ds(out_base + c * CH, CH), :], stage, copy_sem)
            cp.start()
            cp.wait()
            ysum = stage[...] + rbuf[c * CH:(c + 1) * CH, :].astype(jnp.float32)
            ms = jnp.mean(ysum * ysum, axis=-1, keepdims=True)
            out_stage[...] = ysum * lax.rsqrt(ms + 1e-6) * g_ref[...]
            cp = pltpu.make_async_copy(
                out_stage, out_ref.at[pl.ds(c * CH, CH), :], copy_sem)
            cp.start()
            cp.wait()

    return pl.pallas_call(
        body,
        out_shape=jax.ShapeDtypeStruct((HALF, D), jnp.float32),
        in_specs=[pl.BlockSpec(memory_space=pltpu.MemorySpace.ANY),
                  pl.BlockSpec(memory_space=pltpu.MemorySpace.VMEM)],
        out_specs=pl.BlockSpec(memory_space=pltpu.MemorySpace.ANY),
        scratch_shapes=[
            pltpu.VMEM((HALF, D), jnp.bfloat16),
            pltpu.VMEM((QTR, D), jnp.bfloat16),
            pltpu.VMEM((CH, D), jnp.float32),
            pltpu.VMEM((CH, D), jnp.float32),
            pltpu.SemaphoreType.DMA,
            pltpu.SemaphoreType.DMA,
            pltpu.SemaphoreType.DMA,
            pltpu.SemaphoreType.DMA,
            pltpu.SemaphoreType.DMA,
        ],
        compiler_params=pltpu.CompilerParams(collective_id=0),
    )(p2d, g2d)


# baseline (device time: 514247 ns/iter reference)
import jax
import jax.numpy as jnp
from jax import lax
from jax.experimental import pallas as pl
from jax.experimental.pallas import tpu as pltpu

M = 8192
D = 4096
HALF = M // 2
QTR = HALF // 2
CH = 256


def kernel(partial, gamma):
    p2d = partial.reshape(M, D)
    g2d = gamma.reshape(1, D)

    def body(p_ref, g_ref, out_ref, rbuf, send_buf, stage, out_stage,
             send_sem_y, recv_sem_y, send_sem_x, recv_sem_x, copy_sem):
        my_x = lax.axis_index("x")
        my_y = lax.axis_index("y")
        y_nbr = (my_x, 1 - my_y)
        x_nbr = (1 - my_x, my_y)

        barrier = pltpu.get_barrier_semaphore()
        for nbr in (y_nbr, x_nbr):
            pl.semaphore_signal(barrier, inc=1, device_id=nbr,
                                device_id_type=pl.DeviceIdType.MESH)
        pl.semaphore_wait(barrier, 2)

        send_base = (1 - my_y) * HALF + my_x * QTR
        for c in range(QTR // CH):
            cp = pltpu.make_async_copy(
                p_ref.at[pl.ds(send_base + c * CH, CH), :], stage, copy_sem)
            cp.start()
            cp.wait()
            send_buf[pl.ds(c * CH, CH), :] = stage[...].astype(jnp.bfloat16)

        rdma_y = pltpu.make_async_remote_copy(
            src_ref=send_buf,
            dst_ref=rbuf.at[pl.ds(my_x * QTR, QTR), :],
            send_sem=send_sem_y, recv_sem=recv_sem_y,
            device_id=y_nbr, device_id_type=pl.DeviceIdType.MESH)
        rdma_y.start()
        rdma_y.wait()

        rdma_x = pltpu.make_async_remote_copy(
            src_ref=rbuf.at[pl.ds(my_x * QTR, QTR), :],
            dst_ref=rbuf.at[pl.ds(my_x * QTR, QTR), :],
            send_sem=send_sem_x, recv_sem=recv_sem_x,
            device_id=x_nbr, device_id_type=pl.DeviceIdType.MESH)
        rdma_x.start()
        rdma_x.wait()

        out_base = my_y * HALF
        for c in range(HALF // CH):
            cp = pltpu.make_async_copy(
                p_ref.at[pl.ds(out_base + c * CH, CH), :], stage, copy_sem)
            cp.start()
            cp.wait()
            ysum = stage[...] + rbuf[c * CH:(c + 1) * CH, :].astype(jnp.float32)
            ms = jnp.mean(ysum * ysum, axis=-1, keepdims=True)
            out_stage[...] = ysum * lax.rsqrt(ms + 1e-6) * g_ref[...]
            cp = pltpu.make_async_copy(
                out_stage, out_ref.at[pl.ds(c * CH, CH), :], copy_sem)
            cp.start()
            cp.wait()

    return pl.pallas_call(
        body,
        out_shape=jax.ShapeDtypeStruct((HALF, D), jnp.float32),
        in_specs=[pl.BlockSpec(memory_space=pl.ANY),
                  pl.BlockSpec(memory_space=pltpu.MemorySpace.VMEM)],
        out_specs=pl.BlockSpec(memory_space=pl.ANY),
        scratch_shapes=[
            pltpu.VMEM((HALF, D), jnp.bfloat16),
            pltpu.VMEM((QTR, D), jnp.bfloat16),
            pltpu.VMEM((CH, D), jnp.float32),
            pltpu.VMEM((CH, D), jnp.float32),
            pltpu.SemaphoreType.DMA,
            pltpu.SemaphoreType.DMA,
            pltpu.SemaphoreType.DMA,
            pltpu.SemaphoreType.DMA,
            pltpu.SemaphoreType.DMA,
        ],
        compiler_params=pltpu.CompilerParams(
            collective_id=0, vmem_limit_bytes=62 * 1024 * 1024),
    )(p2d, g2d)


# device time: 260654 ns/iter; 1.9729x vs baseline; 1.9729x over previous
import jax
import jax.numpy as jnp
from jax import lax
from jax.experimental import pallas as pl
from jax.experimental.pallas import tpu as pltpu

M = 8192
D = 4096
HALF = M // 2
QTR = HALF // 2
CH = 256
NC = QTR // CH


def kernel(partial, gamma):
    p2d = partial.reshape(M, D)
    g2d = gamma.reshape(1, D)

    def body(p_ref, g_ref, out_ref, rbuf, send_buf, stage, out_stage,
             send_sems_y, recv_sems_y, send_sems_x, recv_sems_x, copy_sem):
        my_x = lax.axis_index("x")
        my_y = lax.axis_index("y")
        y_nbr = (my_x, 1 - my_y)
        x_nbr = (1 - my_x, my_y)

        barrier = pltpu.get_barrier_semaphore()
        for nbr in (y_nbr, x_nbr):
            pl.semaphore_signal(barrier, inc=1, device_id=nbr,
                                device_id_type=pl.DeviceIdType.MESH)
        pl.semaphore_wait(barrier, 2)

        send_base = (1 - my_y) * HALF + my_x * QTR
        y_rdmas = []
        for c in range(NC):
            cp = pltpu.make_async_copy(
                p_ref.at[pl.ds(send_base + c * CH, CH), :], stage, copy_sem)
            cp.start()
            cp.wait()
            send_buf[pl.ds(c * CH, CH), :] = stage[...].astype(jnp.bfloat16)
            rdma = pltpu.make_async_remote_copy(
                src_ref=send_buf.at[pl.ds(c * CH, CH), :],
                dst_ref=rbuf.at[pl.ds(my_x * QTR + c * CH, CH), :],
                send_sem=send_sems_y.at[c], recv_sem=recv_sems_y.at[c],
                device_id=y_nbr, device_id_type=pl.DeviceIdType.MESH)
            rdma.start()
            y_rdmas.append(rdma)

        def compute_chunk(p_off):
            cp = pltpu.make_async_copy(
                p_ref.at[pl.ds(my_y * HALF + p_off, CH), :], stage, copy_sem)
            cp.start()
            cp.wait()
            ysum = stage[...] + rbuf[pl.ds(p_off, CH), :].astype(jnp.float32)
            ms = jnp.mean(ysum * ysum, axis=-1, keepdims=True)
            out_stage[...] = ysum * lax.rsqrt(ms + 1e-6) * g_ref[...]
            cp = pltpu.make_async_copy(
                out_stage, out_ref.at[pl.ds(p_off, CH), :], copy_sem)
            cp.start()
            cp.wait()

        x_rdmas = []
        for c in range(NC):
            y_rdmas[c].wait_recv()
            rdma = pltpu.make_async_remote_copy(
                src_ref=rbuf.at[pl.ds(my_x * QTR + c * CH, CH), :],
                dst_ref=rbuf.at[pl.ds(my_x * QTR + c * CH, CH), :],
                send_sem=send_sems_x.at[c], recv_sem=recv_sems_x.at[c],
                device_id=x_nbr, device_id_type=pl.DeviceIdType.MESH)
            rdma.start()
            x_rdmas.append(rdma)
            compute_chunk(my_x * QTR + c * CH)
            if c >= 1:
                x_rdmas[c - 1].wait_recv()
                compute_chunk((1 - my_x) * QTR + (c - 1) * CH)
        x_rdmas[NC - 1].wait_recv()
        compute_chunk((1 - my_x) * QTR + (NC - 1) * CH)

        for c in range(NC):
            y_rdmas[c].wait_send()
            x_rdmas[c].wait_send()

    return pl.pallas_call(
        body,
        out_shape=jax.ShapeDtypeStruct((HALF, D), jnp.float32),
        in_specs=[pl.BlockSpec(memory_space=pl.ANY),
                  pl.BlockSpec(memory_space=pltpu.MemorySpace.VMEM)],
        out_specs=pl.BlockSpec(memory_space=pl.ANY),
        scratch_shapes=[
            pltpu.VMEM((HALF, D), jnp.bfloat16),
            pltpu.VMEM((QTR, D), jnp.bfloat16),
            pltpu.VMEM((CH, D), jnp.float32),
            pltpu.VMEM((CH, D), jnp.float32),
            pltpu.SemaphoreType.DMA((NC,)),
            pltpu.SemaphoreType.DMA((NC,)),
            pltpu.SemaphoreType.DMA((NC,)),
            pltpu.SemaphoreType.DMA((NC,)),
            pltpu.SemaphoreType.DMA,
        ],
        compiler_params=pltpu.CompilerParams(
            collective_id=0, vmem_limit_bytes=62 * 1024 * 1024),
    )(p2d, g2d)


# device time: 246589 ns/iter; 2.0854x vs baseline; 1.0570x over previous
import jax
import jax.numpy as jnp
from jax import lax
from jax.experimental import pallas as pl
from jax.experimental.pallas import tpu as pltpu

M = 8192
D = 4096
HALF = M // 2
QTR = HALF // 2
CH = 128
NC = QTR // CH


def kernel(partial, gamma):
    p2d = partial.reshape(M, D)
    g2d = gamma.reshape(1, D)

    def body(p_ref, g_ref, out_ref, rbuf, send_buf, stages, outs,
             send_sems_y, recv_sems_y, send_sems_x, recv_sems_x,
             in_sems, out_sems, stage_sem):
        my_x = lax.axis_index("x")
        my_y = lax.axis_index("y")
        y_nbr = (my_x, 1 - my_y)
        x_nbr = (1 - my_x, my_y)

        barrier = pltpu.get_barrier_semaphore()
        for nbr in (y_nbr, x_nbr):
            pl.semaphore_signal(barrier, inc=1, device_id=nbr,
                                device_id_type=pl.DeviceIdType.MESH)
        pl.semaphore_wait(barrier, 2)

        send_base = (1 - my_y) * HALF + my_x * QTR
        y_rdmas = []

        def start_stage_load(c):
            cp = pltpu.make_async_copy(
                p_ref.at[pl.ds(send_base + c * CH, CH), :],
                stages.at[c % 2], in_sems.at[c % 2])
            cp.start()
            return cp

        pending = start_stage_load(0)
        for c in range(NC):
            nxt = start_stage_load(c + 1) if c + 1 < NC else None
            pending.wait()
            send_buf[pl.ds(c * CH, CH), :] = stages[c % 2].astype(jnp.bfloat16)
            pending = nxt
            rdma = pltpu.make_async_remote_copy(
                src_ref=send_buf.at[pl.ds(c * CH, CH), :],
                dst_ref=rbuf.at[pl.ds(my_x * QTR + c * CH, CH), :],
                send_sem=send_sems_y.at[c], recv_sem=recv_sems_y.at[c],
                device_id=y_nbr, device_id_type=pl.DeviceIdType.MESH)
            rdma.start()
            y_rdmas.append(rdma)

        entries = []
        for c in range(NC):
            entries.append(("y", c))
            if c >= 1:
                entries.append(("x", c - 1))
        entries.append(("x", NC - 1))

        def off(kind, c):
            half = my_x * QTR if kind == "y" else (1 - my_x) * QTR
            return half + c * CH

        def start_local_load(k):
            kind, c = entries[k]
            cp = pltpu.make_async_copy(
                p_ref.at[pl.ds(my_y * HALF + off(kind, c), CH), :],
                stages.at[k % 2], in_sems.at[k % 2])
            cp.start()
            return cp

        x_rdmas = []
        out_cps = [None, None]
        load = start_local_load(0)
        for k, (kind, c) in enumerate(entries):
            nxt = start_local_load(k + 1) if k + 1 < len(entries) else None
            if kind == "y":
                y_rdmas[c].wait_recv()
                xr = pltpu.make_async_remote_copy(
                    src_ref=rbuf.at[pl.ds(my_x * QTR + c * CH, CH), :],
                    dst_ref=rbuf.at[pl.ds(my_x * QTR + c * CH, CH), :],
                    send_sem=send_sems_x.at[c], recv_sem=recv_sems_x.at[c],
                    device_id=x_nbr, device_id_type=pl.DeviceIdType.MESH)
                xr.start()
                x_rdmas.append(xr)
            else:
                x_rdmas[c].wait_recv()
            load.wait()
            ysum = stages[k % 2] + rbuf[pl.ds(off(kind, c), CH), :].astype(
                jnp.float32)
            ms = jnp.mean(ysum * ysum, axis=-1, keepdims=True)
            if out_cps[k % 2] is not None:
                out_cps[k % 2].wait()
            outs[k % 2] = ysum * lax.rsqrt(ms + 1e-6) * g_ref[...]
            cp = pltpu.make_async_copy(
                outs.at[k % 2], out_ref.at[pl.ds(off(kind, c), CH), :],
                out_sems.at[k % 2])
            cp.start()
            out_cps[k % 2] = cp
            load = nxt
        for cp in out_cps:
            cp.wait()

        for c in range(NC):
            y_rdmas[c].wait_send()
            x_rdmas[c].wait_send()

    return pl.pallas_call(
        body,
        out_shape=jax.ShapeDtypeStruct((HALF, D), jnp.float32),
        in_specs=[pl.BlockSpec(memory_space=pl.ANY),
                  pl.BlockSpec(memory_space=pltpu.MemorySpace.VMEM)],
        out_specs=pl.BlockSpec(memory_space=pl.ANY),
        scratch_shapes=[
            pltpu.VMEM((HALF, D), jnp.bfloat16),
            pltpu.VMEM((QTR, D), jnp.bfloat16),
            pltpu.VMEM((2, CH, D), jnp.float32),
            pltpu.VMEM((2, CH, D), jnp.float32),
            pltpu.SemaphoreType.DMA((NC,)),
            pltpu.SemaphoreType.DMA((NC,)),
            pltpu.SemaphoreType.DMA((NC,)),
            pltpu.SemaphoreType.DMA((NC,)),
            pltpu.SemaphoreType.DMA((2,)),
            pltpu.SemaphoreType.DMA((2,)),
            pltpu.SemaphoreType.DMA,
        ],
        compiler_params=pltpu.CompilerParams(
            collective_id=0, vmem_limit_bytes=62 * 1024 * 1024),
    )(p2d, g2d)
